# Initial kernel scaffold; baseline (speedup 1.0000x reference)
import jax
import jax.numpy as jnp
from jax.experimental import pallas as pl


def kernel(x, edge_index, W_emb, b_emb, W_theta, b_theta, W_phi, b_phi, W_e1, b_e1, W_e2, b_e2, W_ih, b_ih, W_hh, b_hh, W_out, b_out):
    raise NotImplementedError("write your pallas kernel here")


# speedup vs baseline: 3.7355x; 3.7355x over previous
import functools

import jax
import jax.numpy as jnp
import numpy as np
from jax import lax
from jax.experimental import pallas as pl
from jax.experimental.pallas import tpu as pltpu
from jax.experimental.pallas import tpu_sc as plsc

N_NODES = 10000
N_EDGES = 160000
H = 32
N_STEPS = 3

_NC = 2
_NS = 16
_NW = _NC * _NS
_CHUNK = 125
_GROUP = 8
_NCHUNK = N_EDGES // _CHUNK
_CH_PER_W = _NCHUNK // _NW
_NGROUP = _CH_PER_W // _GROUP
_ROWS_PER_SUB = N_NODES // _NS

_sc_mesh = functools.partial(
    plsc.VectorSubcoreMesh,
    core_axis_name="c", subcore_axis_name="s", num_cores=_NC, num_subcores=_NS)


def _gather_body(tab_ref, idx_ref, out_ref, idxg_v, rows_v, sem):
    c = lax.axis_index("c")
    s = lax.axis_index("s")
    w = s * _NC + c
    for k in range(_NGROUP):
        j0 = w * _CH_PER_W + k * _GROUP
        pltpu.sync_copy(idx_ref.at[pl.ds(j0, _GROUP)], idxg_v)
        descs = []
        for b in range(_GROUP):
            descs.append(pltpu.async_copy(
                tab_ref.at[idxg_v.at[b]], rows_v.at[b], sem))
        for d in descs:
            d.wait()
        pltpu.sync_copy(rows_v, out_ref.at[pl.ds(j0, _GROUP)])


def _sc_gather(tab, idx2):
    out = pl.kernel(
        _gather_body,
        out_type=jax.ShapeDtypeStruct((_NCHUNK, _CHUNK, H), jnp.float32),
        mesh=_sc_mesh(),
        scratch_types=[
            pltpu.VMEM((_GROUP, _CHUNK), jnp.int32),
            pltpu.VMEM((_GROUP, _CHUNK, H), jnp.float32),
            pltpu.SemaphoreType.DMA,
        ],
        compiler_params=pltpu.CompilerParams(use_tc_tiling_on_sc=False),
    )(tab, idx2)
    return out.reshape(N_EDGES, H)


def _scatter_body(m_ref, dst_ref, zeros_ref, out_ref, idxg_v, m_v, sem, acc_sh):
    c = lax.axis_index("c")
    s = lax.axis_index("s")
    w = s * _NC + c
    r0 = s * _ROWS_PER_SUB
    pltpu.sync_copy(zeros_ref.at[s], acc_sh.at[pl.ds(r0, _ROWS_PER_SUB)])
    plsc.subcore_barrier()
    for k in range(_NGROUP):
        j0 = w * _CH_PER_W + k * _GROUP
        pltpu.sync_copy(dst_ref.at[pl.ds(j0, _GROUP)], idxg_v)
        pltpu.sync_copy(m_ref.at[pl.ds(j0, _GROUP)], m_v)
        for b in range(_GROUP):
            pltpu.sync_copy(m_v.at[b], acc_sh.at[idxg_v.at[b]], add=True)
    plsc.subcore_barrier()
    pltpu.sync_copy(acc_sh.at[pl.ds(r0, _ROWS_PER_SUB)], out_ref.at[c, s])


def _sc_scatter(m3, dst2, zeros3):
    out = pl.kernel(
        _scatter_body,
        out_type=jax.ShapeDtypeStruct((_NC, _NS, _ROWS_PER_SUB, H),
                                      jnp.float32),
        mesh=_sc_mesh(),
        scratch_types=[
            pltpu.VMEM((_GROUP, _CHUNK), jnp.int32),
            pltpu.VMEM((_GROUP, _CHUNK, H), jnp.float32),
            pltpu.SemaphoreType.DMA,
            pltpu.VMEM_SHARED((N_NODES, H), jnp.float32),
        ],
        compiler_params=pltpu.CompilerParams(use_tc_tiling_on_sc=False),
    )(m3, dst2, zeros3)
    return out.reshape(_NC * N_NODES, H)


_BN = 1000
_BE = 2000


def _emb_body(x_ref, w_ref, b_ref, o_ref):
    o_ref[:] = jnp.dot(x_ref[:], w_ref[:],
                       preferred_element_type=jnp.float32) + b_ref[:]


def _emb(x, W_emb, b_emb):
    g = N_NODES // _BN
    return pl.pallas_call(
        _emb_body,
        grid=(g,),
        in_specs=[
            pl.BlockSpec((_BN, 128), lambda i: (i, 0)),
            pl.BlockSpec((128, H), lambda i: (0, 0)),
            pl.BlockSpec((1, H), lambda i: (0, 0)),
        ],
        out_specs=pl.BlockSpec((_BN, H), lambda i: (i, 0)),
        out_shape=jax.ShapeDtypeStruct((N_NODES, H), jnp.float32),
    )(x, W_emb, b_emb)


def _edge_mlp_body(hs_ref, hd_ref, wth_ref, wph_ref, bsum_ref, we1_ref,
                   be1_ref, we2_ref, be2_ref, t_ref, r_ref, ew_ref, m1_ref):
    hs = hs_ref[:]
    hd = hd_ref[:]
    he = jnp.maximum(
        jnp.dot(hd - hs, wth_ref[:], preferred_element_type=jnp.float32)
        + jnp.dot(hs, wph_ref[:], preferred_element_type=jnp.float32)
        + bsum_ref[:], 0.0)
    a = jnp.maximum(
        jnp.dot(he, we1_ref[:], preferred_element_type=jnp.float32)
        + be1_ref[:], 0.0)
    z = jnp.dot(a.astype(jnp.bfloat16), we2_ref[:],
                preferred_element_type=jnp.float32) + be2_ref[:]
    ewb = jnp.tanh(z).astype(jnp.bfloat16)
    ew_ref[:] = ewb
    hst = jnp.dot(hs.astype(jnp.bfloat16), t_ref[:],
                  preferred_element_type=jnp.float32).astype(jnp.bfloat16)
    m1_ref[:] = jnp.dot(ewb * hst, r_ref[:],
                        preferred_element_type=jnp.float32)


def _edge_mlp(h_src, h_dst, W_theta, W_phi, bsum, W_e1, b_e1, W_e2b, b_e2,
              T, R):
    g = N_EDGES // _BE
    return pl.pallas_call(
        _edge_mlp_body,
        grid=(g,),
        in_specs=[
            pl.BlockSpec((_BE, H), lambda i: (i, 0)),
            pl.BlockSpec((_BE, H), lambda i: (i, 0)),
            pl.BlockSpec((H, H), lambda i: (0, 0)),
            pl.BlockSpec((H, H), lambda i: (0, 0)),
            pl.BlockSpec((1, H), lambda i: (0, 0)),
            pl.BlockSpec((H, 128), lambda i: (0, 0)),
            pl.BlockSpec((1, 128), lambda i: (0, 0)),
            pl.BlockSpec((128, H * H), lambda i: (0, 0)),
            pl.BlockSpec((1, H * H), lambda i: (0, 0)),
            pl.BlockSpec((H, H * H), lambda i: (0, 0)),
            pl.BlockSpec((H * H, H), lambda i: (0, 0)),
        ],
        out_specs=[
            pl.BlockSpec((_BE, H * H), lambda i: (i, 0)),
            pl.BlockSpec((_BE, H), lambda i: (i, 0)),
        ],
        out_shape=[
            jax.ShapeDtypeStruct((N_EDGES, H * H), jnp.bfloat16),
            jax.ShapeDtypeStruct((N_EDGES, H), jnp.float32),
        ],
    )(h_src, h_dst, W_theta, W_phi, bsum, W_e1, b_e1, W_e2b, b_e2, T, R)


def _msg_body(ew_ref, hs_ref, t_ref, r_ref, m_ref):
    hst = jnp.dot(hs_ref[:].astype(jnp.bfloat16), t_ref[:],
                  preferred_element_type=jnp.float32).astype(jnp.bfloat16)
    m_ref[:] = jnp.dot(ew_ref[:] * hst, r_ref[:],
                       preferred_element_type=jnp.float32)


def _msg(ew, hs, T, R):
    g = N_EDGES // _BE
    return pl.pallas_call(
        _msg_body,
        grid=(g,),
        in_specs=[
            pl.BlockSpec((_BE, H * H), lambda i: (i, 0)),
            pl.BlockSpec((_BE, H), lambda i: (i, 0)),
            pl.BlockSpec((H, H * H), lambda i: (0, 0)),
            pl.BlockSpec((H * H, H), lambda i: (0, 0)),
        ],
        out_specs=pl.BlockSpec((_BE, H), lambda i: (i, 0)),
        out_shape=jax.ShapeDtypeStruct((N_EDGES, H), jnp.float32),
    )(ew, hs, T, R)


def _gru_body(a0_ref, a1_ref, h_ref, wih_ref, bih_ref, whh_ref, bhh_ref,
              o_ref):
    agg = a0_ref[:] + a1_ref[:]
    h = h_ref[:]
    gi = jnp.dot(agg, wih_ref[:], preferred_element_type=jnp.float32) \
        + bih_ref[:]
    gh = jnp.dot(h, whh_ref[:], preferred_element_type=jnp.float32) \
        + bhh_ref[:]
    r = jax.nn.sigmoid(gi[:, :H] + gh[:, :H])
    z = jax.nn.sigmoid(gi[:, H:2 * H] + gh[:, H:2 * H])
    n = jnp.tanh(gi[:, 2 * H:] + r * gh[:, 2 * H:])
    o_ref[:] = (1.0 - z) * n + z * h


def _gru(aggp, h, W_ihT, b_ih, W_hhT, b_hh):
    g = N_NODES // _BN
    return pl.pallas_call(
        _gru_body,
        grid=(g,),
        in_specs=[
            pl.BlockSpec((_BN, H), lambda i: (i, 0)),
            pl.BlockSpec((_BN, H), lambda i: (i + N_NODES // _BN, 0)),
            pl.BlockSpec((_BN, H), lambda i: (i, 0)),
            pl.BlockSpec((H, 3 * H), lambda i: (0, 0)),
            pl.BlockSpec((1, 3 * H), lambda i: (0, 0)),
            pl.BlockSpec((H, 3 * H), lambda i: (0, 0)),
            pl.BlockSpec((1, 3 * H), lambda i: (0, 0)),
        ],
        out_specs=pl.BlockSpec((_BN, H), lambda i: (i, 0)),
        out_shape=jax.ShapeDtypeStruct((N_NODES, H), jnp.float32),
    )(aggp, aggp, h, W_ihT, b_ih, W_hhT, b_hh)


def _out_body(h_ref, w_ref, b_ref, o_ref):
    o_ref[:] = jnp.tanh(
        jnp.dot(h_ref[:], w_ref[:], preferred_element_type=jnp.float32)
        + b_ref[:])


def _out_proj(h, W_out, b_out):
    g = N_NODES // _BN
    return pl.pallas_call(
        _out_body,
        grid=(g,),
        in_specs=[
            pl.BlockSpec((_BN, H), lambda i: (i, 0)),
            pl.BlockSpec((H, H), lambda i: (0, 0)),
            pl.BlockSpec((1, H), lambda i: (0, 0)),
        ],
        out_specs=pl.BlockSpec((_BN, H), lambda i: (i, 0)),
        out_shape=jax.ShapeDtypeStruct((N_NODES, H), jnp.float32),
    )(h, W_out, b_out.reshape(1, H))


def kernel(x, edge_index, W_emb, b_emb, W_theta, b_theta, W_phi, b_phi,
           W_e1, b_e1, W_e2, b_e2, W_ih, b_ih, W_hh, b_hh, W_out, b_out):
    src2 = edge_index[0].reshape(_NCHUNK, _CHUNK)
    dst2 = edge_index[1].reshape(_NCHUNK, _CHUNK)

    T = jnp.asarray(np.concatenate([np.eye(H, dtype=np.float32)] * H, axis=1),
                    dtype=jnp.bfloat16)
    R = jnp.asarray(np.repeat(np.eye(H, dtype=np.float32), H, axis=0),
                    dtype=jnp.bfloat16)

    bsum = (b_theta + b_phi).reshape(1, H)
    W_e2b = W_e2.astype(jnp.bfloat16)
    zeros3 = jnp.zeros((_NS, _ROWS_PER_SUB, H), jnp.float32)

    h = _emb(x, W_emb, b_emb.reshape(1, H))
    h_src = _sc_gather(h, src2)
    h_dst = _sc_gather(h, dst2)
    ew, m = _edge_mlp(h_src, h_dst, W_theta, W_phi, bsum, W_e1,
                      b_e1.reshape(1, 128), W_e2b, b_e2.reshape(1, H * H),
                      T, R)
    W_ihT = W_ih.T
    W_hhT = W_hh.T
    b_ih2 = b_ih.reshape(1, 3 * H)
    b_hh2 = b_hh.reshape(1, 3 * H)
    for step in range(N_STEPS):
        if step > 0:
            hs = _sc_gather(h, src2)
            m = _msg(ew, hs, T, R)
        aggp = _sc_scatter(m.reshape(_NCHUNK, _CHUNK, H), dst2, zeros3)
        h = _gru(aggp, h, W_ihT, b_ih2, W_hhT, b_hh2)
    return _out_proj(h, W_out, b_out)
